# baseline (speedup 1.0000x reference)
import jax
import jax.numpy as jnp
from jax import lax
from jax.experimental import pallas as pl
from jax.experimental.pallas import tpu as pltpu
from jax.experimental.pallas import tpu_sc as plsc

_NUM_CHARS = 128
_L = 20
_E = 64
_EP = _E // 2
_WORDS = 256 * 50
_NW = 32
_WPT = _WORDS // _NW
_SPT = _WPT // 50
_GROUPS = _WPT // 16
_OUT_D = 3 * _E


def _tree_sum(vs):
    while len(vs) > 1:
        nxt = [vs[i] + vs[i + 1] for i in range(0, len(vs) - 1, 2)]
        if len(vs) % 2:
            nxt.append(vs[-1])
        vs = nxt
    return vs[0]


def _sc_body(sntcs_hbm, w2_hbm, out_hbm, chars_v, w2_v, out_v):
    wid = lax.axis_index("s") * 2 + lax.axis_index("c")
    pltpu.sync_copy(sntcs_hbm.at[pl.ds(wid * (_WPT * _L), _WPT * _L)], chars_v)
    pltpu.sync_copy(w2_hbm, w2_v)

    lane = lax.iota(jnp.int32, 16)

    @plsc.parallel_loop(0, _GROUPS, unroll=1)
    def group(g):
        widx = lane + g * 16
        b_v = widx // 50
        w_v = widx - b_v * 50
        cbase = widx * _L
        c = [plsc.load_gather(chars_v, [cbase + l]) for l in range(_L)]

        m = c[0]
        a = jnp.zeros((16,), jnp.int32)
        for l in range(1, _L):
            gt = c[l] > m
            a = jnp.where(gt, l, a)
            m = jnp.where(gt, c[l], m)
        wl = jnp.maximum(a - 1, 0)
        ends = plsc.load_gather(chars_v, [cbase + wl])

        rows = [jnp.where(wl == 0, 0, c[0])]
        rows += [jnp.where(wl == l, 0, c[l]) for l in range(1, _L - 1)]
        rows.append(ends)
        cl32 = [r * _EP for r in rows]

        @plsc.parallel_loop(0, _EP, unroll=8)
        def jbody(j):
            j16 = j & 15
            dmm = (j - j16) + ((j16 + lane) & 15)
            first_w = plsc.load_gather(w2_v, [cl32[0] + dmm])
            bow_w = _tree_sum([
                plsc.bitcast(plsc.load_gather(w2_v, [cl32[l] + dmm]), jnp.bfloat16)
                for l in range(1, _L - 1)
            ])
            last_w = plsc.load_gather(w2_v, [cl32[_L - 1] + dmm])
            f_a, f_b = plsc.unpack(
                plsc.bitcast(first_w, jnp.bfloat16), format=plsc.PackFormat.INTERLEAVED)
            s_a, s_b = plsc.unpack(bow_w, format=plsc.PackFormat.INTERLEAVED)
            l_a, l_b = plsc.unpack(
                plsc.bitcast(last_w, jnp.bfloat16), format=plsc.PackFormat.INTERLEAVED)
            da = dmm * 2
            plsc.store_scatter(out_v, [b_v, w_v, da], f_a)
            plsc.store_scatter(out_v, [b_v, w_v, da + 1], f_b)
            plsc.store_scatter(out_v, [b_v, w_v, da + _E], s_a)
            plsc.store_scatter(out_v, [b_v, w_v, da + (_E + 1)], s_b)
            plsc.store_scatter(out_v, [b_v, w_v, da + 2 * _E], l_a)
            plsc.store_scatter(out_v, [b_v, w_v, da + (2 * _E + 1)], l_b)

    lax.fori_loop

    pltpu.sync_copy(out_v, out_hbm.at[pl.ds(wid * _SPT, _SPT)])


def kernel(sntcs, W):
    s_flat = sntcs.reshape(-1).astype(jnp.int32)
    wb = W.T.astype(jnp.bfloat16)
    w2 = jax.lax.bitcast_convert_type(
        wb.reshape(_NUM_CHARS, _EP, 2), jnp.int32).reshape(-1)
    mesh = plsc.VectorSubcoreMesh(core_axis_name="c", subcore_axis_name="s")
    run = pl.kernel(
        _sc_body,
        mesh=mesh,
        compiler_params=pltpu.CompilerParams(needs_layout_passes=False),
        out_type=jax.ShapeDtypeStruct((256, 50, _OUT_D), jnp.float32),
        scratch_types=[
            pltpu.VMEM((_WPT * _L,), jnp.int32),
            pltpu.VMEM((_NUM_CHARS * _EP,), jnp.int32),
            pltpu.VMEM((_SPT, 50, _OUT_D), jnp.float32),
        ],
    )
    return run(s_flat, w2)


# speedup vs baseline: 1.2089x; 1.2089x over previous
import jax
import jax.numpy as jnp
from jax import lax
from jax.experimental import pallas as pl
from jax.experimental.pallas import tpu as pltpu
from jax.experimental.pallas import tpu_sc as plsc

_NUM_CHARS = 128
_L = 20
_E = 64
_EP = _E // 2
_WORDS = 256 * 50
_NW = 32
_WPT = _WORDS // _NW
_SPT = _WPT // 50
_GROUPS = _WPT // 16
_OUT_D = 3 * _E


def _tree_sum(vs):
    while len(vs) > 1:
        nxt = [vs[i] + vs[i + 1] for i in range(0, len(vs) - 1, 2)]
        if len(vs) % 2:
            nxt.append(vs[-1])
        vs = nxt
    return vs[0]


def _sc_body(sntcs_hbm, w2_hbm, out_hbm, chars_v, w2_v, out_v):
    wid = lax.axis_index("s") * 2 + lax.axis_index("c")
    pltpu.sync_copy(sntcs_hbm.at[pl.ds(wid * _SPT, _SPT)], chars_v)
    pltpu.sync_copy(w2_hbm, w2_v)

    lane = lax.iota(jnp.int32, 16)

    def group(g, carry):
        widx = lane + g * 16
        b_v = widx // 50
        w_v = widx - b_v * 50
        inb = w_v * _L
        c = [plsc.load_gather(chars_v, [b_v, inb + l]) for l in range(_L)]

        m = c[0]
        a = jnp.zeros((16,), jnp.int32)
        for l in range(1, _L):
            gt = c[l] > m
            a = jnp.where(gt, l, a)
            m = jnp.where(gt, c[l], m)
        wl = jnp.maximum(a - 1, 0)
        ends = plsc.load_gather(chars_v, [b_v, inb + wl])

        rows = [jnp.where(wl == 0, 0, c[0])]
        rows += [jnp.where(wl == l, 0, c[l]) for l in range(1, _L - 1)]
        rows.append(ends)
        cl32 = [r * _EP for r in rows]

        @plsc.parallel_loop(0, _EP, unroll=8)
        def jbody(j):
            j16 = j & 15
            dmm = (j - j16) + ((j16 + lane) & 15)
            first_w = plsc.load_gather(w2_v, [cl32[0] + dmm])
            bow_w = _tree_sum([
                plsc.bitcast(plsc.load_gather(w2_v, [cl32[l] + dmm]), jnp.bfloat16)
                for l in range(1, _L - 1)
            ])
            last_w = plsc.load_gather(w2_v, [cl32[_L - 1] + dmm])
            f_a, f_b = plsc.unpack(
                plsc.bitcast(first_w, jnp.bfloat16), format=plsc.PackFormat.INTERLEAVED)
            s_a, s_b = plsc.unpack(bow_w, format=plsc.PackFormat.INTERLEAVED)
            l_a, l_b = plsc.unpack(
                plsc.bitcast(last_w, jnp.bfloat16), format=plsc.PackFormat.INTERLEAVED)
            da = dmm * 2
            plsc.store_scatter(out_v, [b_v, w_v, da], f_a)
            plsc.store_scatter(out_v, [b_v, w_v, da + 1], f_b)
            plsc.store_scatter(out_v, [b_v, w_v, da + _E], s_a)
            plsc.store_scatter(out_v, [b_v, w_v, da + (_E + 1)], s_b)
            plsc.store_scatter(out_v, [b_v, w_v, da + 2 * _E], l_a)
            plsc.store_scatter(out_v, [b_v, w_v, da + (2 * _E + 1)], l_b)

        return carry

    lax.fori_loop(0, _GROUPS, group, 0)
    pltpu.sync_copy(out_v, out_hbm.at[pl.ds(wid * _SPT, _SPT)])


def kernel(sntcs, W):
    s2d = sntcs.reshape(256, 50 * _L).astype(jnp.int32)
    wb = W.T.astype(jnp.bfloat16)
    w2 = jax.lax.bitcast_convert_type(
        wb.reshape(_NUM_CHARS, _EP, 2), jnp.int32).reshape(-1)
    mesh = plsc.VectorSubcoreMesh(core_axis_name="c", subcore_axis_name="s")
    run = pl.kernel(
        _sc_body,
        mesh=mesh,
        compiler_params=pltpu.CompilerParams(needs_layout_passes=False),
        out_type=jax.ShapeDtypeStruct((256, 50, _OUT_D), jnp.float32),
        scratch_types=[
            pltpu.VMEM((_SPT, 50 * _L), jnp.int32),
            pltpu.VMEM((_NUM_CHARS * _EP,), jnp.int32),
            pltpu.VMEM((_SPT, 50, _OUT_D), jnp.float32),
        ],
    )
    return run(s2d, w2)
